# baseline (speedup 1.0000x reference)
import functools

import jax
import jax.numpy as jnp
from jax import lax
from jax.experimental import pallas as pl
from jax.experimental.pallas import tpu as pltpu
from jax.experimental.pallas import tpu_sc as plsc

B = 16384
F = 26
D = 32
V = 1000000
N = B * F

_INFO = plsc.get_sparse_core_info()
NC = _INFO.num_cores
NS = _INFO.num_subcores
NW = NC * NS
L = 16
PER_W = N // NW
CHUNK = 128
NCHUNK = PER_W // CHUNK
NBUF = 4
NGROUP = NCHUNK // NBUF
KBLK = CHUNK // L

assert PER_W * NW == N and NCHUNK * CHUNK == PER_W and NGROUP * NBUF == NCHUNK

_mesh = plsc.VectorSubcoreMesh(core_axis_name="c", subcore_axis_name="s")


@functools.partial(
    pl.kernel,
    mesh=_mesh,
    out_type=jax.ShapeDtypeStruct((N, 128), jnp.float32),
    compiler_params=pltpu.CompilerParams(use_tc_tiling_on_sc=True),
    scratch_types=[
        pltpu.VMEM((NCHUNK, CHUNK), jnp.int32),
        pltpu.VMEM((NBUF, CHUNK), jnp.int32),
        pltpu.VMEM((NBUF, CHUNK, 128), jnp.float32),
        pltpu.SemaphoreType.DMA((NBUF,)),
        pltpu.SemaphoreType.DMA((NBUF,)),
    ],
)
def _gather_kernel(x_hbm, s_hbm, out_hbm, idx_v, glist_v, rows_v, gsem, wsem):
    wid = lax.axis_index("s") * NC + lax.axis_index("c")
    base = wid * PER_W

    pltpu.sync_copy(x_hbm.at[pl.ds(wid * NCHUNK, NCHUNK)], idx_v)

    def start_gather(j, b):
        for k in range(KBLK):
            iv = idx_v[j, pl.ds(k * L, L)]
            glist_v[b, pl.ds(k * L, L)] = lax.shift_right_logical(iv, 2)
        pltpu.make_async_copy(
            s_hbm.at[glist_v.at[b]], rows_v.at[b], gsem.at[b]
        ).start()

    def wait_gather(b):
        pltpu.make_async_copy(
            s_hbm.at[glist_v.at[b]], rows_v.at[b], gsem.at[b]
        ).wait()

    def start_write(j, b):
        pltpu.make_async_copy(
            rows_v.at[b], out_hbm.at[pl.ds(base + j * CHUNK, CHUNK)], wsem.at[b]
        ).start()

    def wait_write(j, b):
        pltpu.make_async_copy(
            rows_v.at[b], out_hbm.at[pl.ds(base + j * CHUNK, CHUNK)], wsem.at[b]
        ).wait()

    for b in range(NBUF):
        start_gather(b, b)

    def group(g, _):
        for b in range(NBUF):
            j = g * NBUF + b
            wait_gather(b)
            start_write(j, b)
        for b in range(NBUF):
            j = g * NBUF + b
            nj = j + NBUF

            @pl.when(nj < NCHUNK)
            def _():
                wait_write(j, b)
                start_gather(nj, b)
        return _

    lax.fori_loop(0, NGROUP, group, None)

    for b in range(NBUF):
        wait_write(NCHUNK - NBUF + b, b)


def kernel(x, table):
    s = table.reshape(V // 4, 128)
    flat = x.reshape(NW * NCHUNK, CHUNK)
    wide = _gather_kernel(flat, s)
    q = jnp.bitwise_and(x.reshape(N), 3)
    cols = q[:, None] * 32 + jnp.arange(32, dtype=jnp.int32)[None, :]
    out = jnp.take_along_axis(wide, cols, axis=1)
    return out.reshape(B, F, D)
